# Initial kernel scaffold; baseline (speedup 1.0000x reference)
import jax
import jax.numpy as jnp
from jax.experimental import pallas as pl


def kernel(concept_ids, fiber_memory):
    raise NotImplementedError("write your pallas kernel here")


# speedup vs baseline: 1.8741x; 1.8741x over previous
import functools

import jax
import jax.numpy as jnp
from jax import lax
from jax.experimental import pallas as pl
from jax.experimental.pallas import tpu as pltpu
from jax.experimental.pallas import tpu_sc as plsc

NUM_CONCEPTS = 1000000
D = 64
GRP = 128
KG = 5
RB = KG * GRP
NB = 40
B_PER_W = NB * RB
NW = 32
TOTAL = NW * B_PER_W


def _make_gather_kernel():
    info = plsc.get_sparse_core_info()
    nc, ns = info.num_cores, info.num_subcores
    assert nc * ns == NW

    mesh = plsc.VectorSubcoreMesh(core_axis_name="c", subcore_axis_name="s")

    @functools.partial(
        pl.kernel,
        mesh=mesh,
        compiler_params=pltpu.CompilerParams(use_tc_tiling_on_sc=False),
        out_type=jax.ShapeDtypeStruct((TOTAL, D), jnp.float32),
        scratch_types=[
            pltpu.VMEM((NB * KG, GRP), jnp.int32),
            pltpu.VMEM((RB, D), jnp.float32),
            pltpu.VMEM((RB, D), jnp.float32),
            pltpu.SemaphoreType.DMA,
            pltpu.SemaphoreType.DMA,
            pltpu.SemaphoreType.DMA,
            pltpu.SemaphoreType.DMA,
        ],
    )
    def gather_kernel(idx_hbm, table_hbm, out_hbm,
                      idx_v, rows0, rows1, g0, g1, s0, s1):
        wid = lax.axis_index("s") * nc + lax.axis_index("c")
        base = wid * B_PER_W

        pltpu.sync_copy(idx_hbm.at[wid], idx_v)

        bufs = ((rows0, g0, s0), (rows1, g1, s1))

        def fire_gathers(i, rows, gsem):
            cps = [
                pltpu.async_copy(
                    table_hbm.at[idx_v.at[i * KG + j]],
                    rows.at[pl.ds(j * GRP, GRP)],
                    gsem,
                )
                for j in range(KG)
            ]
            for cp in cps:
                cp.wait()

        def fire_store(i, rows, ssem):
            pltpu.async_copy(rows, out_hbm.at[pl.ds(base + i * RB, RB)], ssem)

        def wait_store(i, rows, ssem):
            pltpu.make_async_copy(
                rows, out_hbm.at[pl.ds(base + i * RB, RB)], ssem
            ).wait()

        for b, (rows, gsem, ssem) in enumerate(bufs):
            fire_gathers(b, rows, gsem)
            fire_store(b, rows, ssem)

        def body(k, carry):
            i0 = 2 * k
            for b, (rows, gsem, ssem) in enumerate(bufs):
                i = i0 + b
                wait_store(i - 2, rows, ssem)
                fire_gathers(i, rows, gsem)
                fire_store(i, rows, ssem)
            return carry

        lax.fori_loop(1, NB // 2, body, 0)

        for b, (rows, gsem, ssem) in enumerate(bufs):
            wait_store(NB - 2 + b, rows, ssem)

    return gather_kernel


def kernel(concept_ids, fiber_memory):
    bsz, seq = concept_ids.shape
    idx = concept_ids.astype(jnp.int32).reshape(NW, NB * KG, GRP)
    out = _make_gather_kernel()(idx, fiber_memory)
    return out.reshape(bsz, seq, D)


# speedup vs baseline: 1.8763x; 1.0012x over previous
import functools

import jax
import jax.numpy as jnp
from jax import lax
from jax.experimental import pallas as pl
from jax.experimental.pallas import tpu as pltpu
from jax.experimental.pallas import tpu_sc as plsc

NUM_CONCEPTS = 1000000
D = 64
GRP = 128
KG = 5
RB = KG * GRP
NB = 40
B_PER_W = NB * RB
NW = 32
TOTAL = NW * B_PER_W


def _make_gather_kernel():
    info = plsc.get_sparse_core_info()
    nc, ns = info.num_cores, info.num_subcores
    assert nc * ns == NW

    mesh = plsc.VectorSubcoreMesh(core_axis_name="c", subcore_axis_name="s")

    @functools.partial(
        pl.kernel,
        mesh=mesh,
        compiler_params=pltpu.CompilerParams(use_tc_tiling_on_sc=False),
        out_type=jax.ShapeDtypeStruct((TOTAL, D), jnp.float32),
        scratch_types=[
            pltpu.VMEM((NB * KG, GRP), jnp.int32),
            pltpu.VMEM((RB, D), jnp.float32),
            pltpu.VMEM((RB, D), jnp.float32),
            pltpu.SemaphoreType.DMA,
            pltpu.SemaphoreType.DMA,
            pltpu.SemaphoreType.DMA,
            pltpu.SemaphoreType.DMA,
        ],
    )
    def gather_kernel(idx_hbm, table_hbm, out_hbm,
                      idx_v, rows0, rows1, g0, g1, s0, s1):
        wid = lax.axis_index("s") * nc + lax.axis_index("c")
        base = wid * B_PER_W

        pltpu.sync_copy(idx_hbm.at[wid], idx_v)

        bufs = ((rows0, g0, s0), (rows1, g1, s1))

        def fire_gathers(i, rows, gsem):
            for j in range(KG):
                pltpu.async_copy(
                    table_hbm.at[idx_v.at[i * KG + j]],
                    rows.at[pl.ds(j * GRP, GRP)],
                    gsem,
                )

        def wait_gathers(i, rows, gsem):
            for j in range(KG):
                pltpu.make_async_copy(
                    table_hbm.at[idx_v.at[i * KG + j]],
                    rows.at[pl.ds(j * GRP, GRP)],
                    gsem,
                ).wait()

        def fire_store(i, rows, ssem):
            pltpu.async_copy(rows, out_hbm.at[pl.ds(base + i * RB, RB)], ssem)

        def wait_store(i, rows, ssem):
            pltpu.make_async_copy(
                rows, out_hbm.at[pl.ds(base + i * RB, RB)], ssem
            ).wait()

        fire_gathers(0, *bufs[0][:2])
        fire_gathers(1, *bufs[1][:2])
        wait_gathers(0, *bufs[0][:2])
        fire_store(0, *bufs[0][::2])

        def body(k, carry):
            i0 = 2 * k + 1
            for b in range(2):
                i = i0 + b
                cur = bufs[(1 + b) % 2]
                nxt = bufs[b % 2]
                wait_store(i - 1, nxt[0], nxt[2])
                fire_gathers(i + 1, nxt[0], nxt[1])
                wait_gathers(i, cur[0], cur[1])
                fire_store(i, cur[0], cur[2])
            return carry

        lax.fori_loop(0, (NB - 2) // 2, body, 0)

        last = bufs[(NB - 1) % 2]
        wait_gathers(NB - 1, last[0], last[1])
        fire_store(NB - 1, last[0], last[2])
        wait_store(NB - 2, *bufs[(NB - 2) % 2][::2])
        wait_store(NB - 1, *last[::2])

    return gather_kernel


def kernel(concept_ids, fiber_memory):
    bsz, seq = concept_ids.shape
    idx = concept_ids.astype(jnp.int32).reshape(NW, NB * KG, GRP)
    out = _make_gather_kernel()(idx, fiber_memory)
    return out.reshape(bsz, seq, D)


# speedup vs baseline: 2.4325x; 1.2965x over previous
import functools

import jax
import jax.numpy as jnp
from jax import lax
from jax.experimental import pallas as pl
from jax.experimental.pallas import tpu as pltpu
from jax.experimental.pallas import tpu_sc as plsc

NUM_CONCEPTS = 1000000
D = 64
S = 50
NW = 32
SL = 8
B_PER_W = 512
NSLAB = B_PER_W // SL
ROWS_PER_W = B_PER_W * S
ROWS_PER_SLAB = SL * S
L = 16


def _make_gather_kernel():
    info = plsc.get_sparse_core_info()
    nc, ns = info.num_cores, info.num_subcores
    assert nc * ns == NW

    mesh = plsc.VectorSubcoreMesh(core_axis_name="c", subcore_axis_name="s")

    @functools.partial(
        pl.kernel,
        mesh=mesh,
        compiler_params=pltpu.CompilerParams(use_tc_tiling_on_sc=True),
        out_type=jax.ShapeDtypeStruct((16384, S, D), jnp.float32),
        scratch_types=[
            pltpu.VMEM((ROWS_PER_SLAB,), jnp.int32),
            pltpu.VMEM((ROWS_PER_SLAB,), jnp.int32),
            pltpu.VMEM((ROWS_PER_SLAB, D), jnp.float32),
            pltpu.VMEM((ROWS_PER_SLAB, D), jnp.float32),
            pltpu.SemaphoreType.DMA,
            pltpu.SemaphoreType.DMA,
            pltpu.SemaphoreType.DMA,
            pltpu.SemaphoreType.DMA,
            pltpu.SemaphoreType.DMA,
            pltpu.SemaphoreType.DMA,
        ],
    )
    def gather_kernel(idx_hbm, table_hbm, out_hbm,
                      idxc0, idxc1, slab0, slab1, i0, i1, g0, g1, s0, s1):
        wid = lax.axis_index("s") * nc + lax.axis_index("c")
        base_b = wid * B_PER_W
        base_r = wid * ROWS_PER_W

        bufs = ((idxc0, slab0, i0, g0, s0), (idxc1, slab1, i1, g1, s1))

        def fire_idx(g, idxc, isem):
            pltpu.async_copy(
                idx_hbm.at[pl.ds(base_r + g * ROWS_PER_SLAB, ROWS_PER_SLAB)],
                idxc, isem,
            )

        def wait_idx(g, idxc, isem):
            pltpu.make_async_copy(
                idx_hbm.at[pl.ds(base_r + g * ROWS_PER_SLAB, ROWS_PER_SLAB)],
                idxc, isem,
            ).wait()

        def fire_gathers(idxc, slab, gsem):
            def body(c, carry):
                vec = idxc[pl.ds(c * L, L)]
                for u in range(L):
                    i = vec[u]
                    t = c * L + u
                    pltpu.async_copy(
                        table_hbm.at[pl.ds(i, 1)],
                        slab.at[pl.ds(t, 1)],
                        gsem,
                    )
                return carry
            lax.fori_loop(0, ROWS_PER_SLAB // L, body, 0)

        def wait_gathers(slab, gsem):
            def body(c, carry):
                for _ in range(L):
                    pltpu.make_async_copy(
                        table_hbm.at[pl.ds(0, 1)],
                        slab.at[pl.ds(0, 1)],
                        gsem,
                    ).wait()
                return carry
            lax.fori_loop(0, ROWS_PER_SLAB // L, body, 0)

        def fire_stores(g, slab, ssem):
            for bb in range(SL):
                pltpu.async_copy(
                    slab.at[pl.ds(bb * S, S)],
                    out_hbm.at[base_b + g * SL + bb],
                    ssem,
                )

        def wait_stores(g, slab, ssem):
            for bb in range(SL):
                pltpu.make_async_copy(
                    slab.at[pl.ds(bb * S, S)],
                    out_hbm.at[base_b + g * SL + bb],
                    ssem,
                ).wait()

        fire_idx(0, bufs[0][0], bufs[0][2])
        fire_idx(1, bufs[1][0], bufs[1][2])

        def step(k, carry):
            for b, (idxc, slab, isem, gsem, ssem) in enumerate(bufs):
                g = 2 * k + b

                @pl.when(k > 0)
                def _():
                    wait_stores(g - 2, slab, ssem)

                wait_idx(g, idxc, isem)
                fire_gathers(idxc, slab, gsem)

                @pl.when(g + 2 < NSLAB)
                def _():
                    fire_idx(g + 2, idxc, isem)

                wait_gathers(slab, gsem)
                fire_stores(g, slab, ssem)
            return carry

        lax.fori_loop(0, NSLAB // 2, step, 0)

        for b, (idxc, slab, isem, gsem, ssem) in enumerate(bufs):
            wait_stores(NSLAB - 2 + b, slab, ssem)

    return gather_kernel


def kernel(concept_ids, fiber_memory):
    bsz, seq = concept_ids.shape
    idx = concept_ids.astype(jnp.int32).reshape(NW * ROWS_PER_W)
    return _make_gather_kernel()(idx, fiber_memory)
